# baseline (speedup 1.0000x reference)
import functools

import jax
import jax.numpy as jnp
from jax.experimental import pallas as pl
from jax.experimental.pallas import tpu as pltpu
from jax.experimental.pallas import tpu_sc as plsc

_N = 10000
_E = 320000
_D = 128
_NP = 10240
_AW = 144
_CH = 80
_NW = 32
_EPW = _E // _NW
_NCH = _EPW // _CH
_ZROWS = _NP // _NW


def _proj_body(x_ref, w_ref, b_ref, vs_ref, qhi_ref, qlo_ref, khi_ref, klo_ref):
    y = (
        jnp.dot(x_ref[...], w_ref[...], preferred_element_type=jnp.float32)
        + b_ref[...]
    )
    vs_ref[...] = y[:, 256:512]
    qf = y[:, 0:128]
    kf = y[:, 128:256]
    qhi = qf.astype(jnp.bfloat16)
    khi = kf.astype(jnp.bfloat16)
    qhi_ref[...] = qhi
    khi_ref[...] = khi
    qlo_ref[...] = (qf - qhi.astype(jnp.float32)).astype(jnp.bfloat16)
    klo_ref[...] = (kf - khi.astype(jnp.float32)).astype(jnp.bfloat16)


_DN = (((1,), (1,)), ((), ()))


def _gexp_body(qhi_ref, qlo_ref, khi_ref, klo_ref, o_ref):
    qhi, qlo = qhi_ref[...], qlo_ref[...]
    khi, klo = khi_ref[...], klo_ref[...]
    s = jax.lax.dot_general(
        qhi, khi, dimension_numbers=_DN, preferred_element_type=jnp.float32)
    s += jax.lax.dot_general(
        qhi, klo, dimension_numbers=_DN, preferred_element_type=jnp.float32)
    s += jax.lax.dot_general(
        qlo, khi, dimension_numbers=_DN, preferred_element_type=jnp.float32)
    for j2 in range(16):
        o_ref[:, j2, :] = s[:, j2 * 128:(j2 + 1) * 128]


def _final_body(acc_ref, skip_ref, o_ref):
    a = acc_ref[0] + acc_ref[1]
    o_ref[...] = a[:, :_D] / (a[:, _D:_D + 1] + 1e-16) + skip_ref[...]


def _sc_edge_kernel(g_hbm, vp_hbm, src_hbm, dst_hbm, out_hbm,
                    src0, dst0, fidx0, gv0, rows0,
                    src1, dst1, fidx1, gv1, rows1,
                    acc_sh, si0, sg0, sr0, si1, sg1, sr1):
    cid = jax.lax.axis_index("c")
    sid = jax.lax.axis_index("s")
    wid = sid * 2 + cid
    base0 = wid * _EPW

    @pl.loop(0, _CH)
    def _zero_rows(e):
        for j in range(_AW // 16):
            rows0[e, pl.ds(j * 16, 16)] = jnp.zeros((16,), jnp.float32)

    @pl.loop(0, (_NP // 16) // _CH)
    def _zero_acc(c):
        pltpu.sync_copy(rows0, acc_sh.at[pl.ds(sid * (_NP // 16) + c * _CH, _CH)])

    plsc.subcore_barrier()

    def issue_idx(i, sl):
        src_v, dst_v, si = sl[0], sl[1], sl[5]
        base = base0 + i * _CH
        pltpu.async_copy(src_hbm.at[pl.ds(base, _CH)], src_v, si)
        pltpu.async_copy(dst_hbm.at[pl.ds(base, _CH)], dst_v, si)

    def gathers(sl):
        src_v, dst_v, fidx_v, g_v, rows_v, si, sg, sr = sl
        pltpu.make_async_copy(src_hbm.at[pl.ds(0, _CH)], src_v, si).wait()
        pltpu.make_async_copy(dst_hbm.at[pl.ds(0, _CH)], dst_v, si).wait()
        for j in range(_CH // 16):
            s2 = pl.ds(j * 16, 16)
            fidx_v[s2] = dst_v[s2] * _NP + src_v[s2]
        pltpu.async_copy(g_hbm.at[fidx_v], g_v, sg)
        pltpu.async_copy(vp_hbm.at[src_v], rows_v, sr)

    def finish(sl):
        src_v, dst_v, fidx_v, g_v, rows_v, si, sg, sr = sl
        pltpu.make_async_copy(g_hbm.at[fidx_v], g_v, sg).wait()
        pltpu.make_async_copy(vp_hbm.at[src_v], rows_v, sr).wait()

        @pl.loop(0, _CH, step=16)
        def _scale(e0):
            g16 = jnp.exp(g_v[pl.ds(e0, 16)])
            for l in range(16):
                s = g16[l]
                for j in range(_AW // 16):
                    s2 = pl.ds(j * 16, 16)
                    rows_v[e0 + l, s2] = rows_v[e0 + l, s2] * s

        pltpu.sync_copy(rows_v, acc_sh.at[dst_v], add=True)

    s0 = (src0, dst0, fidx0, gv0, rows0, si0, sg0, sr0)
    s1 = (src1, dst1, fidx1, gv1, rows1, si1, sg1, sr1)

    issue_idx(0, s0)
    gathers(s0)
    issue_idx(1, s1)

    @pl.loop(0, _NCH - 1, step=2)
    def _pair(i):
        gathers(s1)
        finish(s0)
        issue_idx(i + 2, s0)
        gathers(s0)
        finish(s1)

        @pl.when(i + 3 < _NCH)
        def _():
            issue_idx(i + 3, s1)

    finish(s0)

    plsc.subcore_barrier()

    @pl.when(sid == 0)
    def _writeout():
        pltpu.sync_copy(acc_sh, out_hbm.at[cid])


def _sc_edge(g_flat, vp, src, dst):
    mesh = plsc.VectorSubcoreMesh(core_axis_name="c", subcore_axis_name="s")
    f = pl.kernel(
        _sc_edge_kernel,
        out_type=jax.ShapeDtypeStruct((2, _NP, _AW), jnp.float32),
        mesh=mesh,
        scratch_types=[
            pltpu.VMEM((_CH,), jnp.int32),
            pltpu.VMEM((_CH,), jnp.int32),
            pltpu.VMEM((_CH,), jnp.int32),
            pltpu.VMEM((_CH,), jnp.float32),
            pltpu.VMEM((_CH, _AW), jnp.float32),
            pltpu.VMEM((_CH,), jnp.int32),
            pltpu.VMEM((_CH,), jnp.int32),
            pltpu.VMEM((_CH,), jnp.int32),
            pltpu.VMEM((_CH,), jnp.float32),
            pltpu.VMEM((_CH, _AW), jnp.float32),
            pltpu.VMEM_SHARED((_NP, _AW), jnp.float32),
            pltpu.SemaphoreType.DMA,
            pltpu.SemaphoreType.DMA,
            pltpu.SemaphoreType.DMA,
            pltpu.SemaphoreType.DMA,
            pltpu.SemaphoreType.DMA,
            pltpu.SemaphoreType.DMA,
        ],
        compiler_params=pltpu.CompilerParams(use_tc_tiling_on_sc=False),
    )
    return f(g_flat, vp, src, dst)


def kernel(x, edge_index, Wq, bq, Wk, bk, Wv, bv, Ws, bs):
    s = 1.0 / jnp.sqrt(jnp.asarray(_D, jnp.float32))
    Wcat = jnp.concatenate([Wq.T * s, Wk.T, Wv.T, Ws.T], axis=1)
    bcat = jnp.concatenate([bq * s, bk, bv, bs]).reshape(1, 512)

    vs, qhi, qlo, khi, klo = pl.pallas_call(
        _proj_body,
        grid=(10,),
        in_specs=[
            pl.BlockSpec((1000, _D), lambda i: (i, 0)),
            pl.BlockSpec((_D, 512), lambda i: (0, 0)),
            pl.BlockSpec((1, 512), lambda i: (0, 0)),
        ],
        out_specs=[
            pl.BlockSpec((1000, 256), lambda i: (i, 0)),
            pl.BlockSpec((1000, _D), lambda i: (i, 0)),
            pl.BlockSpec((1000, _D), lambda i: (i, 0)),
            pl.BlockSpec((1000, _D), lambda i: (i, 0)),
            pl.BlockSpec((1000, _D), lambda i: (i, 0)),
        ],
        out_shape=[
            jax.ShapeDtypeStruct((_N, 256), jnp.float32),
            jax.ShapeDtypeStruct((_N, _D), jnp.bfloat16),
            jax.ShapeDtypeStruct((_N, _D), jnp.bfloat16),
            jax.ShapeDtypeStruct((_N, _D), jnp.bfloat16),
            jax.ShapeDtypeStruct((_N, _D), jnp.bfloat16),
        ],
    )(x, Wcat, bcat)

    v = vs[:, 0:128]
    skip = vs[:, 128:256]

    khip = jnp.zeros((_NP, _D), jnp.bfloat16).at[:_N].set(khi)
    klop = jnp.zeros((_NP, _D), jnp.bfloat16).at[:_N].set(klo)

    g = pl.pallas_call(
        _gexp_body,
        grid=(10, 5),
        in_specs=[
            pl.BlockSpec((1000, _D), lambda i, j: (i, 0)),
            pl.BlockSpec((1000, _D), lambda i, j: (i, 0)),
            pl.BlockSpec((2048, _D), lambda i, j: (j, 0)),
            pl.BlockSpec((2048, _D), lambda i, j: (j, 0)),
        ],
        out_specs=pl.BlockSpec((1000, 16, 128), lambda i, j: (i, j, 0)),
        out_shape=jax.ShapeDtypeStruct((_N, _NP // 128, 128), jnp.float32),
    )(qhi, qlo, khip, klop)

    vp = jnp.concatenate(
        [v, jnp.ones((_N, 1), jnp.float32), jnp.zeros((_N, 15), jnp.float32)],
        axis=1,
    )

    acc = _sc_edge(g.reshape(-1), vp, edge_index[0], edge_index[1])

    out = pl.pallas_call(
        _final_body,
        grid=(10,),
        in_specs=[
            pl.BlockSpec((2, 1000, _AW), lambda i: (0, i, 0)),
            pl.BlockSpec((1000, _D), lambda i: (i, 0)),
        ],
        out_specs=pl.BlockSpec((1000, _D), lambda i: (i, 0)),
        out_shape=jax.ShapeDtypeStruct((_N, _D), jnp.float32),
    )(acc, skip)

    return out


# speedup vs baseline: 1.2046x; 1.2046x over previous
import functools

import jax
import jax.numpy as jnp
from jax.experimental import pallas as pl
from jax.experimental.pallas import tpu as pltpu
from jax.experimental.pallas import tpu_sc as plsc

_N = 10000
_E = 320000
_D = 128
_NP = 10240
_AW = 144
_CH = 80
_NW = 32
_EPW = _E // _NW
_NCH = _EPW // _CH
_ZROWS = _NP // _NW


def _proj_body(x_ref, w_ref, b_ref, o_ref):
    o_ref[...] = (
        jnp.dot(x_ref[...], w_ref[...], preferred_element_type=jnp.float32)
        + b_ref[...]
    )


def _gexp_body(q_ref, k_ref, o_ref):
    s = jax.lax.dot_general(
        q_ref[...], k_ref[...],
        dimension_numbers=(((1,), (1,)), ((), ())),
        preferred_element_type=jnp.float32,
    )
    for j2 in range(16):
        o_ref[:, j2, :] = s[:, j2 * 128:(j2 + 1) * 128]


def _final_body(acc_ref, skip_ref, o_ref):
    a = acc_ref[0] + acc_ref[1]
    o_ref[...] = a[:, :_D] / (a[:, _D:_D + 1] + 1e-16) + skip_ref[...]


def _sc_edge_kernel(g_hbm, vp_hbm, src_hbm, dst_hbm, out_hbm,
                    src0, dst0, fidx0, gv0, rows0,
                    src1, dst1, fidx1, gv1, rows1,
                    acc_sh, si0, sg0, sr0, si1, sg1, sr1):
    cid = jax.lax.axis_index("c")
    sid = jax.lax.axis_index("s")
    wid = sid * 2 + cid
    base0 = wid * _EPW

    @pl.loop(0, _CH)
    def _zero_rows(e):
        for j in range(_AW // 16):
            rows0[e, pl.ds(j * 16, 16)] = jnp.zeros((16,), jnp.float32)

    @pl.loop(0, (_NP // 16) // _CH)
    def _zero_acc(c):
        pltpu.sync_copy(rows0, acc_sh.at[pl.ds(sid * (_NP // 16) + c * _CH, _CH)])

    plsc.subcore_barrier()

    def issue_idx(i, sl):
        src_v, dst_v, si = sl[0], sl[1], sl[5]
        base = base0 + i * _CH
        pltpu.async_copy(src_hbm.at[pl.ds(base, _CH)], src_v, si)
        pltpu.async_copy(dst_hbm.at[pl.ds(base, _CH)], dst_v, si)

    def gathers(sl):
        src_v, dst_v, fidx_v, g_v, rows_v, si, sg, sr = sl
        pltpu.make_async_copy(src_hbm.at[pl.ds(0, _CH)], src_v, si).wait()
        pltpu.make_async_copy(dst_hbm.at[pl.ds(0, _CH)], dst_v, si).wait()
        for j in range(_CH // 16):
            s2 = pl.ds(j * 16, 16)
            fidx_v[s2] = dst_v[s2] * _NP + src_v[s2]
        pltpu.async_copy(g_hbm.at[fidx_v], g_v, sg)
        pltpu.async_copy(vp_hbm.at[src_v], rows_v, sr)

    def finish(sl):
        src_v, dst_v, fidx_v, g_v, rows_v, si, sg, sr = sl
        pltpu.make_async_copy(g_hbm.at[fidx_v], g_v, sg).wait()
        pltpu.make_async_copy(vp_hbm.at[src_v], rows_v, sr).wait()

        @pl.loop(0, _CH, step=16)
        def _scale(e0):
            g16 = jnp.exp(g_v[pl.ds(e0, 16)])
            for l in range(16):
                s = g16[l]
                for j in range(_AW // 16):
                    s2 = pl.ds(j * 16, 16)
                    rows_v[e0 + l, s2] = rows_v[e0 + l, s2] * s

        pltpu.sync_copy(rows_v, acc_sh.at[dst_v], add=True)

    s0 = (src0, dst0, fidx0, gv0, rows0, si0, sg0, sr0)
    s1 = (src1, dst1, fidx1, gv1, rows1, si1, sg1, sr1)

    issue_idx(0, s0)
    gathers(s0)
    issue_idx(1, s1)

    @pl.loop(0, _NCH - 1, step=2)
    def _pair(i):
        gathers(s1)
        finish(s0)
        issue_idx(i + 2, s0)
        gathers(s0)
        finish(s1)

        @pl.when(i + 3 < _NCH)
        def _():
            issue_idx(i + 3, s1)

    finish(s0)

    plsc.subcore_barrier()

    @pl.when(sid == 0)
    def _writeout():
        pltpu.sync_copy(acc_sh, out_hbm.at[cid])


def _sc_edge(g_flat, vp, src, dst):
    mesh = plsc.VectorSubcoreMesh(core_axis_name="c", subcore_axis_name="s")
    f = pl.kernel(
        _sc_edge_kernel,
        out_type=jax.ShapeDtypeStruct((2, _NP, _AW), jnp.float32),
        mesh=mesh,
        scratch_types=[
            pltpu.VMEM((_CH,), jnp.int32),
            pltpu.VMEM((_CH,), jnp.int32),
            pltpu.VMEM((_CH,), jnp.int32),
            pltpu.VMEM((_CH,), jnp.float32),
            pltpu.VMEM((_CH, _AW), jnp.float32),
            pltpu.VMEM((_CH,), jnp.int32),
            pltpu.VMEM((_CH,), jnp.int32),
            pltpu.VMEM((_CH,), jnp.int32),
            pltpu.VMEM((_CH,), jnp.float32),
            pltpu.VMEM((_CH, _AW), jnp.float32),
            pltpu.VMEM_SHARED((_NP, _AW), jnp.float32),
            pltpu.SemaphoreType.DMA,
            pltpu.SemaphoreType.DMA,
            pltpu.SemaphoreType.DMA,
            pltpu.SemaphoreType.DMA,
            pltpu.SemaphoreType.DMA,
            pltpu.SemaphoreType.DMA,
        ],
        compiler_params=pltpu.CompilerParams(use_tc_tiling_on_sc=False),
    )
    return f(g_flat, vp, src, dst)


def kernel(x, edge_index, Wq, bq, Wk, bk, Wv, bv, Ws, bs):
    s = 1.0 / jnp.sqrt(jnp.asarray(_D, jnp.float32))
    Wcat = jnp.concatenate([Wq.T * s, Wk.T, Wv.T, Ws.T], axis=1)
    bcat = jnp.concatenate([bq * s, bk, bv, bs]).reshape(1, 512)

    y = pl.pallas_call(
        _proj_body,
        grid=(10,),
        in_specs=[
            pl.BlockSpec((1000, _D), lambda i: (i, 0)),
            pl.BlockSpec((_D, 512), lambda i: (0, 0)),
            pl.BlockSpec((1, 512), lambda i: (0, 0)),
        ],
        out_specs=pl.BlockSpec((1000, 512), lambda i: (i, 0)),
        out_shape=jax.ShapeDtypeStruct((_N, 512), jnp.float32),
    )(x, Wcat, bcat)

    q = y[:, 0:128]
    k = y[:, 128:256]
    v = y[:, 256:384]
    skip = y[:, 384:512]

    kpad = jnp.zeros((_NP, _D), jnp.float32).at[:_N].set(k)

    g = pl.pallas_call(
        _gexp_body,
        grid=(10, 5),
        in_specs=[
            pl.BlockSpec((1000, _D), lambda i, j: (i, 0)),
            pl.BlockSpec((2048, _D), lambda i, j: (j, 0)),
        ],
        out_specs=pl.BlockSpec((1000, 16, 128), lambda i, j: (i, j, 0)),
        out_shape=jax.ShapeDtypeStruct((_N, _NP // 128, 128), jnp.float32),
    )(q, kpad)

    vp = jnp.concatenate(
        [v, jnp.ones((_N, 1), jnp.float32), jnp.zeros((_N, 15), jnp.float32)],
        axis=1,
    )

    acc = _sc_edge(g.reshape(-1), vp, edge_index[0], edge_index[1])

    out = pl.pallas_call(
        _final_body,
        grid=(10,),
        in_specs=[
            pl.BlockSpec((2, 1000, _AW), lambda i: (0, i, 0)),
            pl.BlockSpec((1000, _D), lambda i: (i, 0)),
        ],
        out_specs=pl.BlockSpec((1000, _D), lambda i: (i, 0)),
        out_shape=jax.ShapeDtypeStruct((_N, _D), jnp.float32),
    )(acc, skip)

    return out


# speedup vs baseline: 1.4827x; 1.2309x over previous
import functools

import jax
import jax.numpy as jnp
from jax.experimental import pallas as pl
from jax.experimental.pallas import tpu as pltpu
from jax.experimental.pallas import tpu_sc as plsc

_N = 10000
_E = 320000
_D = 128
_NP = 10240
_AW = 144
_CH = 80
_NW = 32
_EPW = _E // _NW
_NCH = _EPW // _CH
_ZROWS = _NP // _NW


def _proj_body(x_ref, w_ref, b_ref, o_ref):
    o_ref[...] = (
        jnp.dot(x_ref[...], w_ref[...], preferred_element_type=jnp.float32)
        + b_ref[...]
    )


def _gexp_body(q_ref, k_ref, o_ref):
    s = jax.lax.dot_general(
        q_ref[...], k_ref[...],
        dimension_numbers=(((1,), (1,)), ((), ())),
        preferred_element_type=jnp.float32,
    )
    for j2 in range(16):
        o_ref[j2, :, :] = s[:, j2 * 128:(j2 + 1) * 128]


def _final_body(acc_ref, skip_ref, o_ref):
    a = acc_ref[0] + acc_ref[1]
    o_ref[...] = a[:, :_D] / (a[:, _D:_D + 1] + 1e-16) + skip_ref[...]


def _sc_edge_kernel(g_hbm, vp_hbm, src_hbm, dst_hbm, out_hbm,
                    src0, dst0, fidx0, gv0, rows0,
                    src1, dst1, fidx1, gv1, rows1,
                    acc_sh, si0, sg0, sr0, si1, sg1, sr1):
    cid = jax.lax.axis_index("c")
    sid = jax.lax.axis_index("s")
    wid = sid * 2 + cid
    base0 = wid * _EPW

    @pl.loop(0, _CH)
    def _zero_rows(e):
        for j in range(_AW // 16):
            rows0[e, pl.ds(j * 16, 16)] = jnp.zeros((16,), jnp.float32)

    @pl.loop(0, (_NP // 16) // _CH)
    def _zero_acc(c):
        pltpu.sync_copy(rows0, acc_sh.at[pl.ds(sid * (_NP // 16) + c * _CH, _CH)])

    plsc.subcore_barrier()

    def issue_idx(i, sl):
        src_v, dst_v, si = sl[0], sl[1], sl[5]
        base = base0 + i * _CH
        pltpu.async_copy(src_hbm.at[pl.ds(base, _CH)], src_v, si)
        pltpu.async_copy(dst_hbm.at[pl.ds(base, _CH)], dst_v, si)

    def gathers(sl):
        src_v, dst_v, fidx_v, g_v, rows_v, si, sg, sr = sl
        pltpu.make_async_copy(src_hbm.at[pl.ds(0, _CH)], src_v, si).wait()
        pltpu.make_async_copy(dst_hbm.at[pl.ds(0, _CH)], dst_v, si).wait()
        for j in range(_CH // 16):
            s2 = pl.ds(j * 16, 16)
            s16 = src_v[s2]
            fidx_v[s2] = ((s16 >> 7) * (_N * 128)
                          + (dst_v[s2] << 7) + (s16 & 127))
        pltpu.async_copy(g_hbm.at[fidx_v], g_v, sg)
        pltpu.async_copy(vp_hbm.at[src_v], rows_v, sr)

    def finish(sl):
        src_v, dst_v, fidx_v, g_v, rows_v, si, sg, sr = sl
        pltpu.make_async_copy(g_hbm.at[fidx_v], g_v, sg).wait()
        pltpu.make_async_copy(vp_hbm.at[src_v], rows_v, sr).wait()

        @pl.loop(0, _CH, step=16)
        def _scale(e0):
            g16 = jnp.exp(g_v[pl.ds(e0, 16)])
            for l in range(16):
                s = g16[l]
                for j in range(_AW // 16):
                    s2 = pl.ds(j * 16, 16)
                    rows_v[e0 + l, s2] = rows_v[e0 + l, s2] * s

        pltpu.sync_copy(rows_v, acc_sh.at[dst_v], add=True)

    s0 = (src0, dst0, fidx0, gv0, rows0, si0, sg0, sr0)
    s1 = (src1, dst1, fidx1, gv1, rows1, si1, sg1, sr1)

    issue_idx(0, s0)
    gathers(s0)
    issue_idx(1, s1)

    @pl.loop(0, _NCH - 1, step=2)
    def _pair(i):
        gathers(s1)
        finish(s0)
        issue_idx(i + 2, s0)
        gathers(s0)
        finish(s1)

        @pl.when(i + 3 < _NCH)
        def _():
            issue_idx(i + 3, s1)

    finish(s0)

    plsc.subcore_barrier()

    @pl.when(sid == 0)
    def _writeout():
        pltpu.sync_copy(acc_sh, out_hbm.at[cid])


def _sc_edge(g_flat, vp, src, dst):
    mesh = plsc.VectorSubcoreMesh(core_axis_name="c", subcore_axis_name="s")
    f = pl.kernel(
        _sc_edge_kernel,
        out_type=jax.ShapeDtypeStruct((2, _NP, _AW), jnp.float32),
        mesh=mesh,
        scratch_types=[
            pltpu.VMEM((_CH,), jnp.int32),
            pltpu.VMEM((_CH,), jnp.int32),
            pltpu.VMEM((_CH,), jnp.int32),
            pltpu.VMEM((_CH,), jnp.float32),
            pltpu.VMEM((_CH, _AW), jnp.float32),
            pltpu.VMEM((_CH,), jnp.int32),
            pltpu.VMEM((_CH,), jnp.int32),
            pltpu.VMEM((_CH,), jnp.int32),
            pltpu.VMEM((_CH,), jnp.float32),
            pltpu.VMEM((_CH, _AW), jnp.float32),
            pltpu.VMEM_SHARED((_NP, _AW), jnp.float32),
            pltpu.SemaphoreType.DMA,
            pltpu.SemaphoreType.DMA,
            pltpu.SemaphoreType.DMA,
            pltpu.SemaphoreType.DMA,
            pltpu.SemaphoreType.DMA,
            pltpu.SemaphoreType.DMA,
        ],
        compiler_params=pltpu.CompilerParams(use_tc_tiling_on_sc=False),
    )
    return f(g_flat, vp, src, dst)


def kernel(x, edge_index, Wq, bq, Wk, bk, Wv, bv, Ws, bs):
    s = 1.0 / jnp.sqrt(jnp.asarray(_D, jnp.float32))
    Wcat = jnp.concatenate([Wq.T * s, Wk.T, Wv.T, Ws.T], axis=1)
    bcat = jnp.concatenate([bq * s, bk, bv, bs]).reshape(1, 512)

    y = pl.pallas_call(
        _proj_body,
        grid=(10,),
        in_specs=[
            pl.BlockSpec((1000, _D), lambda i: (i, 0)),
            pl.BlockSpec((_D, 512), lambda i: (0, 0)),
            pl.BlockSpec((1, 512), lambda i: (0, 0)),
        ],
        out_specs=pl.BlockSpec((1000, 512), lambda i: (i, 0)),
        out_shape=jax.ShapeDtypeStruct((_N, 512), jnp.float32),
    )(x, Wcat, bcat)

    q = y[:, 0:128]
    k = y[:, 128:256]
    v = y[:, 256:384]
    skip = y[:, 384:512]

    kpad = jnp.zeros((_NP, _D), jnp.float32).at[:_N].set(k)

    g = pl.pallas_call(
        _gexp_body,
        grid=(10, 5),
        in_specs=[
            pl.BlockSpec((1000, _D), lambda i, j: (i, 0)),
            pl.BlockSpec((2048, _D), lambda i, j: (j, 0)),
        ],
        out_specs=pl.BlockSpec((16, 1000, 128), lambda i, j: (j, i, 0)),
        out_shape=jax.ShapeDtypeStruct((_NP // 128, _N, 128), jnp.float32),
    )(q, kpad)

    vp = jnp.concatenate(
        [v, jnp.ones((_N, 1), jnp.float32), jnp.zeros((_N, 15), jnp.float32)],
        axis=1,
    )

    acc = _sc_edge(g.reshape(-1), vp, edge_index[0], edge_index[1])

    out = pl.pallas_call(
        _final_body,
        grid=(10,),
        in_specs=[
            pl.BlockSpec((2, 1000, _AW), lambda i: (0, i, 0)),
            pl.BlockSpec((1000, _D), lambda i: (i, 0)),
        ],
        out_specs=pl.BlockSpec((1000, _D), lambda i: (i, 0)),
        out_shape=jax.ShapeDtypeStruct((_N, _D), jnp.float32),
    )(acc, skip)

    return out


# speedup vs baseline: 1.6425x; 1.1077x over previous
import jax
import jax.numpy as jnp
from jax.experimental import pallas as pl
from jax.experimental.pallas import tpu as pltpu
from jax.experimental.pallas import tpu_sc as plsc

_N = 10000
_E = 320000
_D = 128
_NP = 10240
_CH = 80
_NW = 32
_EPW = _E // _NW
_NCH = _EPW // _CH


def _proj_body(x_ref, w_ref, b_ref, o_ref):
    o_ref[...] = (
        jnp.dot(x_ref[...], w_ref[...], preferred_element_type=jnp.float32)
        + b_ref[...]
    )


def _gexp_body(q_ref, k_ref, o_ref):
    s = jax.lax.dot_general(
        q_ref[...], k_ref[...],
        dimension_numbers=(((1,), (1,)), ((), ())),
        preferred_element_type=jnp.float32,
    )
    for j2 in range(16):
        o_ref[j2, :, :] = s[:, j2 * 128:(j2 + 1) * 128]


def _final_body(acc_ref, asm_ref, skip_ref, o_ref):
    a = acc_ref[0] + acc_ref[1]
    dn = asm_ref[0, 0] + asm_ref[1, 0]
    dnc = jnp.transpose(dn)
    o_ref[...] = a / (dnc + 1e-16) + skip_ref[...]


def _sc_edge_kernel(g_hbm, vp_hbm, src_hbm, dst_hbm, out_hbm, oas_hbm,
                    src0, dst0, dsc0, fidx0, gv0, rows0,
                    src1, dst1, dsc1, fidx1, gv1, rows1,
                    src2, dst2, dsc2, fidx2, gv2, rows2,
                    zb, acc_sh, asum_sh,
                    si0, sg0, sr0, ss0, sa0,
                    si1, sg1, sr1, ss1, sa1,
                    si2, sg2, sr2, ss2, sa2):
    cid = jax.lax.axis_index("c")
    sid = jax.lax.axis_index("s")
    wid = sid * 2 + cid
    base0 = wid * _EPW

    @pl.loop(0, _CH)
    def _zero_rows(e):
        for j in range(_D // 16):
            rows0[e, pl.ds(j * 16, 16)] = jnp.zeros((16,), jnp.float32)

    @pl.loop(0, _NP // 16, step=16)
    def _zero_zb(e):
        zb[pl.ds(e, 16)] = jnp.zeros((16,), jnp.float32)

    @pl.loop(0, (_NP // 16) // _CH)
    def _zero_acc(c):
        pltpu.sync_copy(rows0, acc_sh.at[pl.ds(sid * (_NP // 16) + c * _CH, _CH)])

    pltpu.sync_copy(zb, asum_sh.at[pl.ds(sid * (_NP // 16), _NP // 16)])
    plsc.subcore_barrier()

    s0 = (src0, dst0, dsc0, fidx0, gv0, rows0, si0, sg0, sr0, ss0, sa0)
    s1 = (src1, dst1, dsc1, fidx1, gv1, rows1, si1, sg1, sr1, ss1, sa1)
    s2 = (src2, dst2, dsc2, fidx2, gv2, rows2, si2, sg2, sr2, ss2, sa2)

    def issue_idx(i, sl):
        base = base0 + i * _CH
        pltpu.async_copy(src_hbm.at[pl.ds(base, _CH)], sl[0], sl[6])
        pltpu.async_copy(dst_hbm.at[pl.ds(base, _CH)], sl[1], sl[6])

    def gathers(sl):
        src_v, dst_v, fidx_v, g_v, rows_v = sl[0], sl[1], sl[3], sl[4], sl[5]
        pltpu.make_async_copy(src_hbm.at[pl.ds(0, _CH)], src_v, sl[6]).wait()
        pltpu.make_async_copy(dst_hbm.at[pl.ds(0, _CH)], dst_v, sl[6]).wait()
        for j in range(_CH // 16):
            s2_ = pl.ds(j * 16, 16)
            s16 = src_v[s2_]
            fidx_v[s2_] = ((s16 >> 7) * (_N * 128)
                           + (dst_v[s2_] << 7) + (s16 & 127))
        pltpu.async_copy(g_hbm.at[fidx_v], g_v, sl[7])
        pltpu.async_copy(vp_hbm.at[src_v], rows_v, sl[8])

    def wait_scatters(sl):
        pltpu.make_async_copy(sl[5], acc_sh.at[sl[2]], sl[9]).wait()
        pltpu.make_async_copy(sl[4], asum_sh.at[sl[2]], sl[10]).wait()

    def finish(sl):
        src_v, dst_v, dsc_v, fidx_v, g_v, rows_v = sl[:6]
        pltpu.make_async_copy(g_hbm.at[fidx_v], g_v, sl[7]).wait()
        pltpu.make_async_copy(vp_hbm.at[src_v], rows_v, sl[8]).wait()

        @pl.loop(0, _CH, step=16)
        def _scale(e0):
            se = pl.ds(e0, 16)
            g16 = jnp.exp(g_v[se])
            g_v[se] = g16
            for l in range(16):
                s = g16[l]
                for j in range(_D // 16):
                    sj = pl.ds(j * 16, 16)
                    rows_v[e0 + l, sj] = rows_v[e0 + l, sj] * s

        for j in range(_CH // 16):
            sj = pl.ds(j * 16, 16)
            dsc_v[sj] = dst_v[sj]
        pltpu.async_copy(rows_v, acc_sh.at[dsc_v], sl[9], add=True)
        pltpu.async_copy(g_v, asum_sh.at[dsc_v], sl[10], add=True)

    issue_idx(0, s0)
    gathers(s0)
    issue_idx(1, s1)

    @pl.loop(0, _NCH - 2, step=3)
    def _tri(i):
        def sub(c, cur, nxt, nx2):
            @pl.when(c >= 2)
            def _():
                wait_scatters(nxt)
            gathers(nxt)
            finish(cur)
            issue_idx(c + 2, nx2)

        sub(i, s0, s1, s2)
        sub(i + 1, s1, s2, s0)
        sub(i + 2, s2, s0, s1)

    wait_scatters(s1)
    gathers(s1)
    finish(s0)
    wait_scatters(s2)
    finish(s1)
    wait_scatters(s0)
    wait_scatters(s1)

    plsc.subcore_barrier()

    @pl.when(sid == 0)
    def _writeout():
        pltpu.sync_copy(acc_sh, out_hbm.at[cid])
        pltpu.sync_copy(asum_sh, oas_hbm.at[cid])


def _sc_edge(g_flat, v, src, dst):
    mesh = plsc.VectorSubcoreMesh(core_axis_name="c", subcore_axis_name="s")
    slot_types = [
        pltpu.VMEM((_CH,), jnp.int32),
        pltpu.VMEM((_CH,), jnp.int32),
        pltpu.VMEM((_CH,), jnp.int32),
        pltpu.VMEM((_CH,), jnp.int32),
        pltpu.VMEM((_CH,), jnp.float32),
        pltpu.VMEM((_CH, _D), jnp.float32),
    ]
    f = pl.kernel(
        _sc_edge_kernel,
        out_type=[
            jax.ShapeDtypeStruct((2, _NP, _D), jnp.float32),
            jax.ShapeDtypeStruct((2, _NP), jnp.float32),
        ],
        mesh=mesh,
        scratch_types=(
            slot_types * 3
            + [
                pltpu.VMEM((_NP // 16,), jnp.float32),
                pltpu.VMEM_SHARED((_NP, _D), jnp.float32),
                pltpu.VMEM_SHARED((_NP,), jnp.float32),
            ]
            + [pltpu.SemaphoreType.DMA] * 15
        ),
        compiler_params=pltpu.CompilerParams(use_tc_tiling_on_sc=False),
    )
    return f(g_flat, v, src, dst)


def kernel(x, edge_index, Wq, bq, Wk, bk, Wv, bv, Ws, bs):
    s = 1.0 / jnp.sqrt(jnp.asarray(_D, jnp.float32))
    Wcat = jnp.concatenate([Wq.T * s, Wk.T, Wv.T, Ws.T], axis=1)
    bcat = jnp.concatenate([bq * s, bk, bv, bs]).reshape(1, 512)

    y = pl.pallas_call(
        _proj_body,
        grid=(10,),
        in_specs=[
            pl.BlockSpec((1000, _D), lambda i: (i, 0)),
            pl.BlockSpec((_D, 512), lambda i: (0, 0)),
            pl.BlockSpec((1, 512), lambda i: (0, 0)),
        ],
        out_specs=pl.BlockSpec((1000, 512), lambda i: (i, 0)),
        out_shape=jax.ShapeDtypeStruct((_N, 512), jnp.float32),
    )(x, Wcat, bcat)

    q = y[:, 0:128]
    k = y[:, 128:256]
    v = y[:, 256:384]
    skip = y[:, 384:512]

    kpad = jnp.zeros((_NP, _D), jnp.float32).at[:_N].set(k)

    g = pl.pallas_call(
        _gexp_body,
        grid=(10, 5),
        in_specs=[
            pl.BlockSpec((1000, _D), lambda i, j: (i, 0)),
            pl.BlockSpec((2048, _D), lambda i, j: (j, 0)),
        ],
        out_specs=pl.BlockSpec((16, 1000, 128), lambda i, j: (j, i, 0)),
        out_shape=jax.ShapeDtypeStruct((_NP // 128, _N, 128), jnp.float32),
    )(q, kpad)

    acc, asm = _sc_edge(g.reshape(-1), v, edge_index[0], edge_index[1])
    asm4 = asm.reshape(2, _NP // 128, 1, 128)
    skip_p = jnp.zeros((_NP, _D), jnp.float32).at[:_N].set(skip)

    out = pl.pallas_call(
        _final_body,
        grid=(_NP // 128,),
        in_specs=[
            pl.BlockSpec((2, 128, _D), lambda i: (0, i, 0)),
            pl.BlockSpec((2, 1, 1, 128), lambda i: (0, i, 0, 0)),
            pl.BlockSpec((128, _D), lambda i: (i, 0)),
        ],
        out_specs=pl.BlockSpec((128, _D), lambda i: (i, 0)),
        out_shape=jax.ShapeDtypeStruct((_NP, _D), jnp.float32),
    )(acc, asm4, skip_p)

    return out[:_N]


# speedup vs baseline: 1.7453x; 1.0626x over previous
import jax
import jax.numpy as jnp
from jax.experimental import pallas as pl
from jax.experimental.pallas import tpu as pltpu
from jax.experimental.pallas import tpu_sc as plsc

_N = 10000
_E = 320000
_D = 128
_NP = 10240
_CH = 80
_NW = 32
_EPW = _E // _NW
_NCH = _EPW // _CH


def _proj_body(x_ref, w_ref, b_ref, o_ref):
    o_ref[...] = (
        jnp.dot(x_ref[...], w_ref[...], preferred_element_type=jnp.float32)
        + b_ref[...]
    )


def _gexp_body(q_ref, k_ref, o_ref):
    s = jax.lax.dot_general(
        q_ref[...], k_ref[...],
        dimension_numbers=(((1,), (1,)), ((), ())),
        preferred_element_type=jnp.float32,
    )
    for j2 in range(16):
        o_ref[j2, :, :] = s[:, j2 * 128:(j2 + 1) * 128]


def _final_body(acc_ref, asm_ref, skip_ref, o_ref):
    a = acc_ref[0] + acc_ref[1]
    dn = asm_ref[0, 0] + asm_ref[1, 0]
    dnc = jnp.transpose(dn)
    o_ref[...] = a / (dnc + 1e-16) + skip_ref[...]


def _sc_edge_kernel(g_hbm, vp_hbm, src_hbm, dst_hbm, out_hbm, oas_hbm,
                    src0, dst0, dsc0, fidx0, gv0, rows0,
                    src1, dst1, dsc1, fidx1, gv1, rows1,
                    src2, dst2, dsc2, fidx2, gv2, rows2,
                    zb, acc_sh, asum_sh,
                    si0, sg0, sr0, ss0, sa0,
                    si1, sg1, sr1, ss1, sa1,
                    si2, sg2, sr2, ss2, sa2):
    cid = jax.lax.axis_index("c")
    sid = jax.lax.axis_index("s")
    wid = sid * 2 + cid
    base0 = wid * _EPW

    @pl.loop(0, _CH)
    def _zero_rows(e):
        for j in range(_D // 16):
            rows0[e, pl.ds(j * 16, 16)] = jnp.zeros((16,), jnp.float32)

    @pl.loop(0, _NP // 16, step=16)
    def _zero_zb(e):
        zb[pl.ds(e, 16)] = jnp.zeros((16,), jnp.float32)

    @pl.loop(0, (_NP // 16) // _CH)
    def _zero_acc(c):
        pltpu.sync_copy(rows0, acc_sh.at[pl.ds(sid * (_NP // 16) + c * _CH, _CH)])

    pltpu.sync_copy(zb, asum_sh.at[pl.ds(sid * (_NP // 16), _NP // 16)])
    plsc.subcore_barrier()

    s0 = (src0, dst0, dsc0, fidx0, gv0, rows0, si0, sg0, sr0, ss0, sa0)
    s1 = (src1, dst1, dsc1, fidx1, gv1, rows1, si1, sg1, sr1, ss1, sa1)
    s2 = (src2, dst2, dsc2, fidx2, gv2, rows2, si2, sg2, sr2, ss2, sa2)

    def issue_idx(i, sl):
        base = base0 + i * _CH
        pltpu.async_copy(src_hbm.at[pl.ds(base, _CH)], sl[0], sl[6])
        pltpu.async_copy(dst_hbm.at[pl.ds(base, _CH)], sl[1], sl[6])

    def gathers(sl):
        src_v, dst_v, fidx_v, g_v, rows_v = sl[0], sl[1], sl[3], sl[4], sl[5]
        pltpu.make_async_copy(src_hbm.at[pl.ds(0, _CH)], src_v, sl[6]).wait()
        pltpu.make_async_copy(dst_hbm.at[pl.ds(0, _CH)], dst_v, sl[6]).wait()
        pltpu.async_copy(vp_hbm.at[src_v], rows_v, sl[8])
        for j in range(_CH // 16):
            s2_ = pl.ds(j * 16, 16)
            s16 = src_v[s2_]
            fidx_v[s2_] = ((s16 >> 7) * (_N * 128)
                           + (dst_v[s2_] << 7) + (s16 & 127))
        pltpu.async_copy(g_hbm.at[fidx_v], g_v, sl[7])

    def wait_scatters(sl):
        pltpu.make_async_copy(sl[5], acc_sh.at[sl[2]], sl[9]).wait()
        pltpu.make_async_copy(sl[4], asum_sh.at[sl[2]], sl[10]).wait()

    def finish(sl):
        src_v, dst_v, dsc_v, fidx_v, g_v, rows_v = sl[:6]
        pltpu.make_async_copy(g_hbm.at[fidx_v], g_v, sl[7]).wait()
        pltpu.make_async_copy(vp_hbm.at[src_v], rows_v, sl[8]).wait()

        @pl.loop(0, _CH, step=16)
        def _scale(e0):
            se = pl.ds(e0, 16)
            g16 = jnp.exp(g_v[se])
            g_v[se] = g16
            for l in range(16):
                s = g16[l]
                for j in range(_D // 16):
                    sj = pl.ds(j * 16, 16)
                    rows_v[e0 + l, sj] = rows_v[e0 + l, sj] * s

        for j in range(_CH // 16):
            sj = pl.ds(j * 16, 16)
            dsc_v[sj] = dst_v[sj]
        pltpu.async_copy(rows_v, acc_sh.at[dsc_v], sl[9], add=True)
        pltpu.async_copy(g_v, asum_sh.at[dsc_v], sl[10], add=True)

    issue_idx(0, s0)
    gathers(s0)
    issue_idx(1, s1)

    @pl.loop(0, _NCH - 2, step=3)
    def _tri(i):
        def sub(c, cur, nxt, nx2):
            @pl.when(c >= 2)
            def _():
                wait_scatters(nxt)
            issue_idx(c + 2, nx2)
            gathers(nxt)
            finish(cur)

        sub(i, s0, s1, s2)
        sub(i + 1, s1, s2, s0)
        sub(i + 2, s2, s0, s1)

    wait_scatters(s1)
    gathers(s1)
    finish(s0)
    wait_scatters(s2)
    finish(s1)
    wait_scatters(s0)
    wait_scatters(s1)

    plsc.subcore_barrier()

    @pl.when(sid == 0)
    def _writeout():
        pltpu.sync_copy(acc_sh, out_hbm.at[cid])
        pltpu.sync_copy(asum_sh, oas_hbm.at[cid])


def _sc_edge(g_flat, v, src, dst):
    mesh = plsc.VectorSubcoreMesh(core_axis_name="c", subcore_axis_name="s")
    slot_types = [
        pltpu.VMEM((_CH,), jnp.int32),
        pltpu.VMEM((_CH,), jnp.int32),
        pltpu.VMEM((_CH,), jnp.int32),
        pltpu.VMEM((_CH,), jnp.int32),
        pltpu.VMEM((_CH,), jnp.float32),
        pltpu.VMEM((_CH, _D), jnp.float32),
    ]
    f = pl.kernel(
        _sc_edge_kernel,
        out_type=[
            jax.ShapeDtypeStruct((2, _NP, _D), jnp.float32),
            jax.ShapeDtypeStruct((2, _NP), jnp.float32),
        ],
        mesh=mesh,
        scratch_types=(
            slot_types * 3
            + [
                pltpu.VMEM((_NP // 16,), jnp.float32),
                pltpu.VMEM_SHARED((_NP, _D), jnp.float32),
                pltpu.VMEM_SHARED((_NP,), jnp.float32),
            ]
            + [pltpu.SemaphoreType.DMA] * 15
        ),
        compiler_params=pltpu.CompilerParams(use_tc_tiling_on_sc=False),
    )
    return f(g_flat, v, src, dst)


def kernel(x, edge_index, Wq, bq, Wk, bk, Wv, bv, Ws, bs):
    s = 1.0 / jnp.sqrt(jnp.asarray(_D, jnp.float32))
    Wcat = jnp.concatenate([Wq.T * s, Wk.T, Wv.T, Ws.T], axis=1)
    bcat = jnp.concatenate([bq * s, bk, bv, bs]).reshape(1, 512)

    y = pl.pallas_call(
        _proj_body,
        grid=(10,),
        in_specs=[
            pl.BlockSpec((1000, _D), lambda i: (i, 0)),
            pl.BlockSpec((_D, 512), lambda i: (0, 0)),
            pl.BlockSpec((1, 512), lambda i: (0, 0)),
        ],
        out_specs=pl.BlockSpec((1000, 512), lambda i: (i, 0)),
        out_shape=jax.ShapeDtypeStruct((_N, 512), jnp.float32),
    )(x, Wcat, bcat)

    q = y[:, 0:128]
    k = y[:, 128:256]
    v = y[:, 256:384]
    skip = y[:, 384:512]

    kpad = jnp.zeros((_NP, _D), jnp.float32).at[:_N].set(k)

    g = pl.pallas_call(
        _gexp_body,
        grid=(10, 5),
        in_specs=[
            pl.BlockSpec((1000, _D), lambda i, j: (i, 0)),
            pl.BlockSpec((2048, _D), lambda i, j: (j, 0)),
        ],
        out_specs=pl.BlockSpec((16, 1000, 128), lambda i, j: (j, i, 0)),
        out_shape=jax.ShapeDtypeStruct((_NP // 128, _N, 128), jnp.float32),
    )(q, kpad)

    acc, asm = _sc_edge(g.reshape(-1), v, edge_index[0], edge_index[1])
    asm4 = asm.reshape(2, _NP // 128, 1, 128)
    skip_p = jnp.zeros((_NP, _D), jnp.float32).at[:_N].set(skip)

    out = pl.pallas_call(
        _final_body,
        grid=(_NP // 128,),
        in_specs=[
            pl.BlockSpec((2, 128, _D), lambda i: (0, i, 0)),
            pl.BlockSpec((2, 1, 1, 128), lambda i: (0, i, 0, 0)),
            pl.BlockSpec((128, _D), lambda i: (i, 0)),
        ],
        out_specs=pl.BlockSpec((128, _D), lambda i: (i, 0)),
        out_shape=jax.ShapeDtypeStruct((_NP, _D), jnp.float32),
    )(acc, asm4, skip_p)

    return out[:_N]
